# Initial kernel scaffold; baseline (speedup 1.0000x reference)
import jax
import jax.numpy as jnp
from jax.experimental import pallas as pl


def kernel(seq):
    raise NotImplementedError("write your pallas kernel here")


# speedup vs baseline: 1.4569x; 1.4569x over previous
import functools

import jax
import jax.numpy as jnp
from jax import lax
from jax.experimental import pallas as pl
from jax.experimental.pallas import tpu as pltpu
from jax.experimental.pallas import tpu_sc as plsc

_LANES = 16
_NC = 2
_NS = 16
_NW = _NC * _NS


def _pick_block(n):
    for b in (8000, 4000, 2000, 1000, 800, 400, 160, 80, 16):
        if n % b == 0:
            return b
    return None


def _body(seq_hbm, out_hbm, in_v, o0, o1, o2, o3, *, n, blk, nblocks, kmax):
    wid = lax.axis_index("s") * _NC + lax.axis_index("c")

    def step(k, carry):
        bid = wid + k * _NW

        @pl.when(bid < nblocks)
        def _():
            base = bid * blk
            pltpu.sync_copy(seq_hbm.at[pl.ds(base, blk)], in_v)

            def inner(i, c2):
                off = pl.multiple_of(i * _LANES, _LANES)
                s = in_v[pl.ds(off, _LANES)]
                one = jnp.full((_LANES,), 1.0, jnp.float32)
                nv = jnp.where(s == 4,
                               jnp.full((_LANES,), 0.25, jnp.float32),
                               jnp.zeros((_LANES,), jnp.float32))
                for c, o in enumerate((o0, o1, o2, o3)):
                    o[pl.ds(off, _LANES)] = jnp.where(s == c, one, nv)
                return c2

            lax.fori_loop(0, blk // _LANES, inner, 0)
            for c, o in enumerate((o0, o1, o2, o3)):
                pltpu.sync_copy(o, out_hbm.at[pl.ds(c * n + base, blk)])

        return carry

    lax.fori_loop(0, kmax, step, 0)


def kernel(seq):
    n = seq.shape[0]
    blk = _pick_block(n)
    nblocks = n // blk
    kmax = -(-nblocks // _NW)
    mesh = plsc.VectorSubcoreMesh(core_axis_name="c", subcore_axis_name="s")
    f = pl.kernel(
        functools.partial(_body, n=n, blk=blk, nblocks=nblocks, kmax=kmax),
        out_type=jax.ShapeDtypeStruct((4 * n,), jnp.float32),
        mesh=mesh,
        scratch_types=[pltpu.VMEM((blk,), jnp.int32)]
        + [pltpu.VMEM((blk,), jnp.float32) for _ in range(4)],
    )
    return f(seq.astype(jnp.int32)).reshape(4, n)


# speedup vs baseline: 1.4616x; 1.0032x over previous
import functools

import jax
import jax.numpy as jnp
from jax import lax
from jax.experimental import pallas as pl
from jax.experimental.pallas import tpu as pltpu
from jax.experimental.pallas import tpu_sc as plsc

_LANES = 16
_NC = 2
_NS = 16
_NW = _NC * _NS


def _pick_block(n):
    for b in (8000, 4000, 2000, 1000, 800, 400, 160, 80, 16):
        if n % b == 0:
            return b
    return None


def _body(seq_hbm, out_hbm, *refs, n, blk, nblocks, kmax):
    ins = refs[0:2]
    outs = (refs[2:6], refs[6:10])
    isems = refs[10:12]
    osems = refs[12:14]

    wid = lax.axis_index("s") * _NC + lax.axis_index("c")

    def bid(k):
        return wid + k * _NW

    def pred(k):
        return bid(k) < nblocks

    def start_in(k):
        pltpu.async_copy(seq_hbm.at[pl.ds(bid(k) * blk, blk)],
                         ins[k % 2], isems[k % 2])

    def wait_in(k):
        pltpu.make_async_copy(seq_hbm.at[pl.ds(bid(k) * blk, blk)],
                              ins[k % 2], isems[k % 2]).wait()

    def start_out(k):
        base = bid(k) * blk
        for c in range(4):
            pltpu.async_copy(outs[k % 2][c],
                             out_hbm.at[pl.ds(c * n + base, blk)],
                             osems[k % 2])

    def wait_out(k):
        base = bid(k) * blk
        for c in range(4):
            pltpu.make_async_copy(outs[k % 2][c],
                                  out_hbm.at[pl.ds(c * n + base, blk)],
                                  osems[k % 2]).wait()

    def compute(k):
        iv = ins[k % 2]
        ov = outs[k % 2]

        def inner(i, c2):
            off = pl.multiple_of(i * _LANES, _LANES)
            s = iv[pl.ds(off, _LANES)]
            one = jnp.full((_LANES,), 1.0, jnp.float32)
            nv = jnp.where(s == 4,
                           jnp.full((_LANES,), 0.25, jnp.float32),
                           jnp.zeros((_LANES,), jnp.float32))
            for c in range(4):
                ov[c][pl.ds(off, _LANES)] = jnp.where(s == c, one, nv)
            return c2

        lax.fori_loop(0, blk // _LANES, inner, 0, unroll=2)

    @pl.when(pred(0))
    def _():
        start_in(0)

    for k in range(kmax):
        if k + 1 < kmax:
            @pl.when(pred(k + 1))
            def _(k=k):
                start_in(k + 1)

        @pl.when(pred(k))
        def _(k=k):
            wait_in(k)
            if k >= 2:
                wait_out(k - 2)
            compute(k)
            start_out(k)

    for k in range(max(0, kmax - 2), kmax):
        @pl.when(pred(k))
        def _(k=k):
            wait_out(k)


def kernel(seq):
    n = seq.shape[0]
    blk = _pick_block(n)
    nblocks = n // blk
    kmax = -(-nblocks // _NW)
    mesh = plsc.VectorSubcoreMesh(core_axis_name="c", subcore_axis_name="s")
    f = pl.kernel(
        functools.partial(_body, n=n, blk=blk, nblocks=nblocks, kmax=kmax),
        out_type=jax.ShapeDtypeStruct((4 * n,), jnp.float32),
        mesh=mesh,
        scratch_types=[pltpu.VMEM((blk,), jnp.int32) for _ in range(2)]
        + [pltpu.VMEM((blk,), jnp.float32) for _ in range(8)]
        + [pltpu.SemaphoreType.DMA for _ in range(4)],
    )
    return f(seq.astype(jnp.int32)).reshape(4, n)


# speedup vs baseline: 1.4798x; 1.0125x over previous
import functools

import jax
import jax.numpy as jnp
from jax import lax
from jax.experimental import pallas as pl
from jax.experimental.pallas import tpu as pltpu
from jax.experimental.pallas import tpu_sc as plsc

_LANES = 16
_NC = 2
_NS = 16
_NW = _NC * _NS


def _pick_block(n):
    for b in (8000, 4000, 2000, 1000, 800, 400, 160, 80, 16):
        if n % b == 0:
            return b
    return None


def _body(seq_hbm, out_hbm, *refs, n, blk, nblocks, kmax):
    ins = refs[0:2]
    outs = (refs[2:6], refs[6:10])
    isems = refs[10:12]
    osems = refs[12:14]

    wid = lax.axis_index("s") * _NC + lax.axis_index("c")

    def bid(k):
        return wid + k * _NW

    def pred(k):
        return bid(k) < nblocks

    def start_in(k):
        pltpu.async_copy(seq_hbm.at[pl.ds(bid(k) * blk, blk)],
                         ins[k % 2], isems[k % 2])

    def wait_in(k):
        pltpu.make_async_copy(seq_hbm.at[pl.ds(bid(k) * blk, blk)],
                              ins[k % 2], isems[k % 2]).wait()

    def start_out(k):
        base = bid(k) * blk
        for c in range(4):
            pltpu.async_copy(outs[k % 2][c],
                             out_hbm.at[pl.ds(c * n + base, blk)],
                             osems[k % 2])

    def wait_out(k):
        base = bid(k) * blk
        for c in range(4):
            pltpu.make_async_copy(outs[k % 2][c],
                                  out_hbm.at[pl.ds(c * n + base, blk)],
                                  osems[k % 2]).wait()

    def compute(k):
        iv = ins[k % 2]
        ov = outs[k % 2]

        @plsc.parallel_loop(0, blk, step=_LANES, unroll=8)
        def inner(i):
            off = pl.multiple_of(i, _LANES)
            s = iv[pl.ds(off, _LANES)]
            one = jnp.full((_LANES,), 1.0, jnp.float32)
            nv = jnp.where(s == 4,
                           jnp.full((_LANES,), 0.25, jnp.float32),
                           jnp.zeros((_LANES,), jnp.float32))
            for c in range(4):
                ov[c][pl.ds(off, _LANES)] = jnp.where(s == c, one, nv)

    @pl.when(pred(0))
    def _():
        start_in(0)

    for k in range(kmax):
        if k + 1 < kmax:
            @pl.when(pred(k + 1))
            def _(k=k):
                start_in(k + 1)

        @pl.when(pred(k))
        def _(k=k):
            wait_in(k)
            if k >= 2:
                wait_out(k - 2)
            compute(k)
            start_out(k)

    for k in range(max(0, kmax - 2), kmax):
        @pl.when(pred(k))
        def _(k=k):
            wait_out(k)


def kernel(seq):
    n = seq.shape[0]
    blk = _pick_block(n)
    nblocks = n // blk
    kmax = -(-nblocks // _NW)
    mesh = plsc.VectorSubcoreMesh(core_axis_name="c", subcore_axis_name="s")
    f = pl.kernel(
        functools.partial(_body, n=n, blk=blk, nblocks=nblocks, kmax=kmax),
        out_type=jax.ShapeDtypeStruct((4 * n,), jnp.float32),
        mesh=mesh,
        scratch_types=[pltpu.VMEM((blk,), jnp.int32) for _ in range(2)]
        + [pltpu.VMEM((blk,), jnp.float32) for _ in range(8)]
        + [pltpu.SemaphoreType.DMA for _ in range(4)],
    )
    return f(seq.astype(jnp.int32)).reshape(4, n)
